# Initial kernel scaffold; baseline (speedup 1.0000x reference)
import jax
import jax.numpy as jnp
from jax.experimental import pallas as pl


def kernel(x, embed_weight, fc1_weight, fc1_bias):
    raise NotImplementedError("write your pallas kernel here")


# speedup vs baseline: 31.0156x; 31.0156x over previous
import functools

import jax
import jax.numpy as jnp
from jax import lax
from jax.experimental import pallas as pl
from jax.experimental.pallas import tpu as pltpu
from jax.experimental.pallas import tpu_sc as plsc

V = 1000000
D = 32
C = 16
B = 4096
L = 200

NC = 2
NS = 16
NW = NC * NS
BPW = B // NW
CH = 100
NCHUNK = BPW * (L // CH)
UNROLL = 10


def _sc_forward(x_resh, embed_weight, fc1_wt, fc1_bias):
    mesh = plsc.VectorSubcoreMesh(core_axis_name="c", subcore_axis_name="s")

    @functools.partial(
        pl.kernel,
        mesh=mesh,
        compiler_params=pltpu.CompilerParams(use_tc_tiling_on_sc=False),
        out_type=jax.ShapeDtypeStruct((B, C), jnp.float32),
        scratch_types=[
            pltpu.VMEM((NCHUNK, CH), jnp.int32),
            pltpu.VMEM((2, CH, D), jnp.float32),
            pltpu.VMEM((BPW, C), jnp.float32),
            pltpu.VMEM((D, C), jnp.float32),
            pltpu.VMEM((C,), jnp.float32),
            pltpu.SemaphoreType.DMA,
        ],
    )
    def body(x_hbm, emb_hbm, wt_hbm, bias_hbm, out_hbm,
             idx_v, rows_v, out_v, wt_v, bias_v, sem):
        wid = lax.axis_index("s") * NC + lax.axis_index("c")
        pltpu.sync_copy(x_hbm.at[wid], idx_v)
        pltpu.sync_copy(wt_hbm, wt_v)
        pltpu.sync_copy(bias_hbm, bias_v)

        def row_body(b, carry):
            cp0 = pltpu.async_copy(emb_hbm.at[idx_v.at[2 * b]], rows_v.at[0], sem)
            cp1 = pltpu.async_copy(emb_hbm.at[idx_v.at[2 * b + 1]], rows_v.at[1], sem)
            cp0.wait()
            cp1.wait()

            acc0 = rows_v[0, 0, 0:16]
            acc1 = rows_v[0, 0, 16:32]

            def red_body(i, accs):
                a0, a1 = accs
                for j in range(UNROLL):
                    l = i * UNROLL + j
                    a0 = jnp.maximum(a0, rows_v[0, l, 0:16])
                    a1 = jnp.maximum(a1, rows_v[0, l, 16:32])
                    a0 = jnp.maximum(a0, rows_v[1, l, 0:16])
                    a1 = jnp.maximum(a1, rows_v[1, l, 16:32])
                return a0, a1

            acc0, acc1 = lax.fori_loop(0, CH // UNROLL, red_body, (acc0, acc1))

            o = bias_v[0:16]
            for d in range(C):
                o = o + acc0[d] * wt_v[d, 0:16]
                o = o + acc1[d] * wt_v[C + d, 0:16]
            out_v[b, 0:16] = o
            return carry

        lax.fori_loop(0, BPW, row_body, 0)
        pltpu.sync_copy(out_v, out_hbm.at[pl.ds(wid * BPW, BPW)])

    return body(x_resh, embed_weight, fc1_wt, fc1_bias)


def kernel(x, embed_weight, fc1_weight, fc1_bias):
    x_resh = x.reshape(NW, NCHUNK, CH)
    fc1_wt = fc1_weight.T
    return _sc_forward(x_resh, embed_weight, fc1_wt, fc1_bias)
